# baseline (speedup 1.0000x reference)
import functools

import jax
import jax.numpy as jnp
from jax import lax
from jax.experimental import pallas as pl
from jax.experimental.pallas import tpu as pltpu
from jax.experimental.pallas import tpu_sc as plsc

R = 2048
N = 50176
H = 224
W = 224
WV = W // 16
K = 128
NW = 32
RPW = R // NW
NV = N // 16
CAP = 8192
L = 16

_MESH = plsc.VectorSubcoreMesh(core_axis_name="c", subcore_axis_name="s")


def _key_from_f32(v):
    k = plsc.bitcast(v, jnp.int32)
    return jnp.where(k < 0, k ^ jnp.int32(0x7FFFFFFF), k)


def _sc_body(x_hbm, pos_hbm, out_hbm, rowbuf, idxbuf, keybuf, outval,
             outidx, posbuf, sem):
    wid = lax.axis_index("s") * 2 + lax.axis_index("c")
    iota = lax.iota(jnp.int32, L)
    zero16 = jnp.zeros((L,), jnp.int32)
    neg_inf = jnp.full((L,), -jnp.inf, jnp.float32)

    @pl.loop(0, RPW)
    def _row_loop(j):
        row = wid * RPW + j
        pltpu.sync_copy(x_hbm.at[row], rowbuf)

        @plsc.parallel_loop(0, H, unroll=2,
                            carry=tuple(neg_inf for _ in range(8)))
        def _maxes(r, carry):
            vs = [rowbuf[r, pl.ds(u * L, L)] for u in range(WV)]
            return tuple(
                jnp.maximum(carry[u],
                            jnp.maximum(vs[u], vs[u + 8]) if u + 8 < WV
                            else vs[u])
                for u in range(8)
            )

        m = list(_maxes)
        for st in (4, 2, 1):
            for u in range(st):
                m[u] = jnp.minimum(m[u], m[u + st])
        t = jnp.min(m[0])
        mx = list(_maxes)
        for st in (4, 2, 1):
            for u in range(st):
                mx[u] = jnp.maximum(mx[u], mx[u + st])
        key_hi = jnp.max(_key_from_f32(mx[0]))
        key_lo = jnp.min(_key_from_f32(m[0]))

        @plsc.parallel_loop(0, H, carry=zero16)
        def _filt(r, ptrv):
            for u in range(WV):
                vals = rowbuf[r, pl.ds(u * L, L)]
                mge = vals >= t
                cum = plsc.cumsum(mge.astype(jnp.int32))
                pos = ptrv + cum - 1
                okw = mge & (pos < CAP)
                plsc.store_scatter(idxbuf, [pos], iota + (r * W + u * L),
                                   mask=okw)
                ptrv = ptrv + plsc.all_reduce_population_count(mge)
            return ptrv

        c_cnt = jnp.minimum(jnp.max(_filt), CAP)
        nv = (c_cnt + (L - 1)) // L

        @plsc.parallel_loop(0, nv)
        def _keys(g):
            lanes = iota + g * L
            lm = lanes < c_cnt
            iv = jnp.where(lm, idxbuf[pl.ds(g * L, L)], 0)
            vals = plsc.load_gather(rowbuf, [iv // W, iv % W])
            key = _key_from_f32(vals)
            keybuf[pl.ds(g * L, L)] = jnp.where(lm, key,
                                                jnp.int32(-0x80000000))

        def _count_gt(pivot):
            @plsc.parallel_loop(0, nv, carry=zero16)
            def _cnt(g, acc):
                kv = keybuf[pl.ds(g * L, L)]
                return acc + plsc.all_reduce_population_count(kv > pivot)
            return jnp.max(_cnt)

        def _bs_cond(s):
            lo, hi = s
            return hi - lo > 1

        def _bs_body(s):
            lo, hi = s
            mid = lo + ((hi - lo) >> 1)
            big = _count_gt(mid) >= K
            return (jnp.where(big, mid, lo), jnp.where(big, hi, mid))

        _, vstar = lax.while_loop(_bs_cond, _bs_body, (key_lo - 1, key_hi))
        g_cnt = _count_gt(vstar)
        b_allow = K - g_cnt

        jsplat = zero16 + j

        @plsc.parallel_loop(0, nv, carry=(zero16, zero16))
        def _sel(g, carry):
            ptrv, tiev = carry
            kv = keybuf[pl.ds(g * L, L)]
            gtm = kv > vstar
            eqm = kv == vstar
            cum_t = tiev + plsc.cumsum(eqm.astype(jnp.int32))
            selm = gtm | (eqm & (cum_t <= b_allow))
            posn = ptrv + plsc.cumsum(selm.astype(jnp.int32)) - 1
            vals = plsc.bitcast(jnp.where(kv < 0, kv ^ jnp.int32(0x7FFFFFFF),
                                          kv), jnp.float32)
            iv = idxbuf[pl.ds(g * L, L)]
            plsc.store_scatter(outidx, [jsplat, posn], iv, mask=selm)
            plsc.store_scatter(outval, [jsplat, posn], vals, mask=selm)
            return (ptrv + plsc.all_reduce_population_count(selm),
                    tiev + plsc.all_reduce_population_count(eqm))

    @pl.loop(0, RPW)
    def _pe_loop(j):
        pltpu.sync_copy(pos_hbm.at[outidx.at[j]], posbuf.at[j])
        ov = outval.at[j]
        pv = posbuf.at[j]

        @pl.loop(0, K // L)
        def _add(u):
            s = pl.ds(u * L, L)
            ov[s] = ov[s] + pv[s]

    pltpu.sync_copy(outval, out_hbm.at[pl.ds(wid * RPW, RPW)])


@jax.jit
def _run(x2, posf):
    f = pl.kernel(
        _sc_body,
        out_type=jax.ShapeDtypeStruct((R, K), jnp.float32),
        mesh=_MESH,
        scratch_types=[
            pltpu.VMEM((H, W), jnp.float32),
            pltpu.VMEM((CAP,), jnp.int32),
            pltpu.VMEM((CAP,), jnp.int32),
            pltpu.VMEM((RPW, K), jnp.float32),
            pltpu.VMEM((RPW, K), jnp.int32),
            pltpu.VMEM((RPW, K), jnp.float32),
            pltpu.SemaphoreType.DMA,
        ],
        compiler_params=pltpu.CompilerParams(needs_layout_passes=False),
    )
    return f(x2, posf)


def kernel(x, pos_enc):
    b, c, h, w = x.shape
    out = _run(x.reshape(b * c, h, w), pos_enc.reshape(h * w))
    return out.reshape(b, c, K)


# speedup vs baseline: 2.1852x; 2.1852x over previous
import functools

import jax
import jax.numpy as jnp
from jax import lax
from jax.experimental import pallas as pl
from jax.experimental.pallas import tpu as pltpu
from jax.experimental.pallas import tpu_sc as plsc

R = 2048
N = 50176
K = 128
NW = 32
RPW = R // NW
NV = N // 16
CAP = 8192
L = 16

_MESH = plsc.VectorSubcoreMesh(core_axis_name="c", subcore_axis_name="s")


def _key_from_f32(v):
    k = plsc.bitcast(v, jnp.int32)
    return jnp.where(k < 0, k ^ jnp.int32(0x7FFFFFFF), k)


def _sc_body(x_hbm, pos_hbm, out_hbm, rowbuf, idxbuf, keybuf, outval,
             outidx, posbuf, sem):
    wid = lax.axis_index("s") * 2 + lax.axis_index("c")
    iota = lax.iota(jnp.int32, L)
    zero16 = jnp.zeros((L,), jnp.int32)
    neg_inf = jnp.full((L,), -jnp.inf, jnp.float32)

    @pl.loop(0, RPW)
    def _row_loop(j):
        row = wid * RPW + j
        pltpu.sync_copy(x_hbm.at[row], rowbuf)

        @pl.loop(0, NV // 8, init_carry=tuple(neg_inf for _ in range(8)), unroll=2)
        def _maxes(i, carry):
            base = i * (8 * L)
            return tuple(
                jnp.maximum(carry[u], rowbuf[pl.ds(base + u * L, L)])
                for u in range(8)
            )

        m = list(_maxes)
        for st in (4, 2, 1):
            for u in range(st):
                m[u] = jnp.minimum(m[u], m[u + st])
        t = jnp.min(m[0])
        mx = list(_maxes)
        for st in (4, 2, 1):
            for u in range(st):
                mx[u] = jnp.maximum(mx[u], mx[u + st])
        key_hi = jnp.max(_key_from_f32(mx[0]))
        key_lo = jnp.min(_key_from_f32(m[0]))

        @plsc.parallel_loop(0, NV, unroll=8, carry=zero16)
        def _filt(v, ptrv):
            vals = rowbuf[pl.ds(v * L, L)]
            mge = vals >= t
            cum = plsc.cumsum(mge.astype(jnp.int32))
            pos = ptrv + cum - 1
            okw = mge & (pos < CAP)
            plsc.store_scatter(idxbuf, [pos], iota + v * L, mask=okw)
            return ptrv + plsc.all_reduce_population_count(mge)

        c_cnt = jnp.minimum(jnp.max(_filt), CAP)
        nv = (c_cnt + (L - 1)) // L

        @plsc.parallel_loop(0, nv)
        def _keys(g):
            lanes = iota + g * L
            lm = lanes < c_cnt
            iv = jnp.where(lm, idxbuf[pl.ds(g * L, L)], 0)
            vals = plsc.load_gather(rowbuf, [iv])
            key = _key_from_f32(vals)
            keybuf[pl.ds(g * L, L)] = jnp.where(lm, key,
                                                jnp.int32(-0x80000000))

        def _count_gt(pivot):
            @plsc.parallel_loop(0, nv, carry=zero16)
            def _cnt(g, acc):
                kv = keybuf[pl.ds(g * L, L)]
                return acc + plsc.all_reduce_population_count(kv > pivot)
            return jnp.max(_cnt)

        def _bs_cond(s):
            lo, hi = s
            return hi - lo > 1

        def _bs_body(s):
            lo, hi = s
            mid = lo + ((hi - lo) >> 1)
            big = _count_gt(mid) >= K
            return (jnp.where(big, mid, lo), jnp.where(big, hi, mid))

        _, vstar = lax.while_loop(_bs_cond, _bs_body, (key_lo - 1, key_hi))
        g_cnt = _count_gt(vstar)
        b_allow = K - g_cnt

        jsplat = zero16 + j

        @plsc.parallel_loop(0, nv, carry=(zero16, zero16))
        def _sel(g, carry):
            ptrv, tiev = carry
            kv = keybuf[pl.ds(g * L, L)]
            gtm = kv > vstar
            eqm = kv == vstar
            cum_t = tiev + plsc.cumsum(eqm.astype(jnp.int32))
            selm = gtm | (eqm & (cum_t <= b_allow))
            posn = ptrv + plsc.cumsum(selm.astype(jnp.int32)) - 1
            vals = plsc.bitcast(jnp.where(kv < 0, kv ^ jnp.int32(0x7FFFFFFF),
                                          kv), jnp.float32)
            iv = idxbuf[pl.ds(g * L, L)]
            plsc.store_scatter(outidx, [jsplat, posn], iv, mask=selm)
            plsc.store_scatter(outval, [jsplat, posn], vals, mask=selm)
            return (ptrv + plsc.all_reduce_population_count(selm),
                    tiev + plsc.all_reduce_population_count(eqm))

    @pl.loop(0, RPW)
    def _pe_loop(j):
        pltpu.sync_copy(pos_hbm.at[outidx.at[j]], posbuf.at[j])
        ov = outval.at[j]
        pv = posbuf.at[j]

        @pl.loop(0, K // L)
        def _add(u):
            s = pl.ds(u * L, L)
            ov[s] = ov[s] + pv[s]

    pltpu.sync_copy(outval, out_hbm.at[pl.ds(wid * RPW, RPW)])


@jax.jit
def _run(x2, posf):
    f = pl.kernel(
        _sc_body,
        out_type=jax.ShapeDtypeStruct((R, K), jnp.float32),
        mesh=_MESH,
        scratch_types=[
            pltpu.VMEM((N,), jnp.float32),
            pltpu.VMEM((CAP,), jnp.int32),
            pltpu.VMEM((CAP,), jnp.int32),
            pltpu.VMEM((RPW, K), jnp.float32),
            pltpu.VMEM((RPW, K), jnp.int32),
            pltpu.VMEM((RPW, K), jnp.float32),
            pltpu.SemaphoreType.DMA,
        ],
        compiler_params=pltpu.CompilerParams(needs_layout_passes=False),
    )
    return f(x2, posf)


def kernel(x, pos_enc):
    b, c, h, w = x.shape
    out = _run(x.reshape(b * c, h * w), pos_enc.reshape(h * w))
    return out.reshape(b, c, K)


# speedup vs baseline: 2.1874x; 1.0010x over previous
import functools

import jax
import jax.numpy as jnp
from jax import lax
from jax.experimental import pallas as pl
from jax.experimental.pallas import tpu as pltpu
from jax.experimental.pallas import tpu_sc as plsc

R = 2048
N = 50176
K = 128
NW = 32
RPW = R // NW
NV = N // 16
CAP = 8192
L = 16

_MESH = plsc.VectorSubcoreMesh(core_axis_name="c", subcore_axis_name="s")


def _key_from_f32(v):
    k = plsc.bitcast(v, jnp.int32)
    return jnp.where(k < 0, k ^ jnp.int32(0x7FFFFFFF), k)


def _sc_body(x_hbm, pos_hbm, out_hbm, rowbuf, idxbuf, keybuf, outval,
             outidx, posbuf, sem):
    wid = lax.axis_index("s") * 2 + lax.axis_index("c")
    iota = lax.iota(jnp.int32, L)
    zero16 = jnp.zeros((L,), jnp.int32)
    neg_inf = jnp.full((L,), -jnp.inf, jnp.float32)

    @pl.loop(0, RPW)
    def _row_loop(j):
        row = wid * RPW + j
        pltpu.sync_copy(x_hbm.at[row], rowbuf)

        @pl.loop(0, NV // 8, init_carry=tuple(neg_inf for _ in range(8)), unroll=2)
        def _maxes(i, carry):
            base = i * (8 * L)
            return tuple(
                jnp.maximum(carry[u], rowbuf[pl.ds(base + u * L, L)])
                for u in range(8)
            )

        m = list(_maxes)
        for st in (4, 2, 1):
            for u in range(st):
                m[u] = jnp.minimum(m[u], m[u + st])
        t = jnp.min(m[0])
        mx = list(_maxes)
        for st in (4, 2, 1):
            for u in range(st):
                mx[u] = jnp.maximum(mx[u], mx[u + st])
        key_hi = jnp.max(_key_from_f32(mx[0]))
        key_lo = jnp.min(_key_from_f32(m[0]))

        @plsc.parallel_loop(0, NV, unroll=8, carry=zero16)
        def _filt(v, ptrv):
            vals = rowbuf[pl.ds(v * L, L)]
            mge = vals >= t
            cum = plsc.cumsum(mge.astype(jnp.int32))
            pos = ptrv + cum - 1
            okw = mge & (pos < CAP)
            plsc.store_scatter(idxbuf, [pos], iota + v * L, mask=okw)
            return ptrv + plsc.all_reduce_population_count(mge)

        c_cnt = jnp.minimum(jnp.max(_filt), CAP)
        nv = (c_cnt + (L - 1)) // L

        @plsc.parallel_loop(0, nv)
        def _keys(g):
            lanes = iota + g * L
            lm = lanes < c_cnt
            iv = jnp.where(lm, idxbuf[pl.ds(g * L, L)], 0)
            vals = plsc.load_gather(rowbuf, [iv])
            key = _key_from_f32(vals)
            keybuf[pl.ds(g * L, L)] = jnp.where(lm, key,
                                                jnp.int32(-0x80000000))

        def _count_gt(pivot):
            @plsc.parallel_loop(0, nv, carry=zero16)
            def _cnt(g, acc):
                kv = keybuf[pl.ds(g * L, L)]
                return acc + plsc.all_reduce_population_count(kv > pivot)
            return jnp.max(_cnt)

        def _bs_cond(s):
            lo, hi = s
            return hi - lo > 1

        def _bs_body(s):
            lo, hi = s
            mid = lo + ((hi - lo) >> 1)
            big = _count_gt(mid) >= K
            return (jnp.where(big, mid, lo), jnp.where(big, hi, mid))

        _, vstar = lax.while_loop(_bs_cond, _bs_body, (key_lo - 1, key_hi))
        g_cnt = _count_gt(vstar)
        b_allow = K - g_cnt

        jsplat = zero16 + j

        @plsc.parallel_loop(0, nv, carry=(zero16, zero16))
        def _sel(g, carry):
            ptrv, tiev = carry
            kv = keybuf[pl.ds(g * L, L)]
            gtm = kv > vstar
            eqm = kv == vstar
            cum_t = tiev + plsc.cumsum(eqm.astype(jnp.int32))
            selm = gtm | (eqm & (cum_t <= b_allow))
            posn = ptrv + plsc.cumsum(selm.astype(jnp.int32)) - 1
            vals = plsc.bitcast(jnp.where(kv < 0, kv ^ jnp.int32(0x7FFFFFFF),
                                          kv), jnp.float32)
            iv = idxbuf[pl.ds(g * L, L)]
            plsc.store_scatter(outidx, [jsplat, posn], iv, mask=selm)
            plsc.store_scatter(outval, [jsplat, posn], vals, mask=selm)
            return (ptrv + plsc.all_reduce_population_count(selm),
                    tiev + plsc.all_reduce_population_count(eqm))

    @pl.loop(0, RPW)
    def _pe_loop(j):
        pltpu.sync_copy(pos_hbm.at[outidx.at[j]], posbuf.at[j])
        ov = outval.at[j]
        pv = posbuf.at[j]

        @pl.loop(0, K // L)
        def _add(u):
            s = pl.ds(u * L, L)
            ov[s] = ov[s] + pv[s]

    pltpu.sync_copy(outval, out_hbm.at[pl.ds(wid * RPW, RPW)])


@jax.jit
def _run(x2, posf):
    f = pl.kernel(
        _sc_body,
        out_type=jax.ShapeDtypeStruct((R, K), jnp.float32),
        mesh=_MESH,
        scratch_types=[
            pltpu.VMEM((N,), jnp.float32),
            pltpu.VMEM((CAP,), jnp.int32),
            pltpu.VMEM((CAP,), jnp.int32),
            pltpu.VMEM((RPW, K), jnp.float32),
            pltpu.VMEM((RPW, K), jnp.int32),
            pltpu.VMEM((RPW, K), jnp.float32),
            pltpu.SemaphoreType.DMA,
        ],
        compiler_params=pltpu.CompilerParams(needs_layout_passes=False, use_tc_tiling_on_sc=True),
    )
    return f(x2, posf)


def kernel(x, pos_enc):
    b, c, h, w = x.shape
    out = _run(x.reshape(b * c, h * w), pos_enc.reshape(h * w))
    return out.reshape(b, c, K)


# speedup vs baseline: 2.3807x; 1.0884x over previous
import functools

import jax
import jax.numpy as jnp
from jax import lax
from jax.experimental import pallas as pl
from jax.experimental.pallas import tpu as pltpu
from jax.experimental.pallas import tpu_sc as plsc

R = 2048
N = 50176
K = 128
NW = 32
RPW = R // NW
NV = N // 16
CAP = 4096
L = 16

_MESH = plsc.VectorSubcoreMesh(core_axis_name="c", subcore_axis_name="s")


def _key_from_f32(v):
    k = plsc.bitcast(v, jnp.int32)
    return jnp.where(k < 0, k ^ jnp.int32(0x7FFFFFFF), k)


def _sc_body(x_hbm, pos_hbm, out_hbm, rowbuf, idxbuf, keybuf, outval,
             outidx, posbuf, sem):
    wid = lax.axis_index("s") * 2 + lax.axis_index("c")
    iota = lax.iota(jnp.int32, L)
    zero16 = jnp.zeros((L,), jnp.int32)
    neg_inf = jnp.full((L,), -jnp.inf, jnp.float32)

    pltpu.async_copy(x_hbm.at[wid * RPW], rowbuf.at[pl.ds(0, N)], sem)

    @pl.loop(0, RPW)
    def _row_loop(j):
        row = wid * RPW + j

        @pl.when(j + 1 < RPW)
        def _():
            pltpu.async_copy(x_hbm.at[row + 1],
                             rowbuf.at[pl.ds(((j + 1) % 2) * N, N)], sem)

        pltpu.make_async_copy(x_hbm.at[row],
                              rowbuf.at[pl.ds((j % 2) * N, N)], sem).wait()
        rb = rowbuf.at[pl.ds((j % 2) * N, N)]

        @pl.loop(0, NV // 8, init_carry=tuple(neg_inf for _ in range(8)), unroll=2)
        def _maxes(i, carry):
            base = i * (8 * L)
            return tuple(
                jnp.maximum(carry[u], rb[pl.ds(base + u * L, L)])
                for u in range(8)
            )

        m = list(_maxes)
        for st in (4, 2, 1):
            for u in range(st):
                m[u] = jnp.minimum(m[u], m[u + st])
        t = jnp.min(m[0])
        mx = list(_maxes)
        for st in (4, 2, 1):
            for u in range(st):
                mx[u] = jnp.maximum(mx[u], mx[u + st])
        key_hi = jnp.max(_key_from_f32(mx[0]))
        key_lo = jnp.min(_key_from_f32(m[0]))

        @plsc.parallel_loop(0, NV, unroll=8, carry=zero16)
        def _filt(v, ptrv):
            vals = rb[pl.ds(v * L, L)]
            mge = vals >= t
            cum = plsc.cumsum(mge.astype(jnp.int32))
            pos = ptrv + cum - 1
            okw = mge & (pos < CAP)
            plsc.store_scatter(idxbuf, [pos], iota + v * L, mask=okw)
            return ptrv + plsc.all_reduce_population_count(mge)

        c_cnt = jnp.minimum(jnp.max(_filt), CAP)
        nv = (c_cnt + (L - 1)) // L

        @plsc.parallel_loop(0, nv)
        def _keys(g):
            lanes = iota + g * L
            lm = lanes < c_cnt
            iv = jnp.where(lm, idxbuf[pl.ds(g * L, L)], 0)
            vals = plsc.load_gather(rb, [iv])
            key = _key_from_f32(vals)
            keybuf[pl.ds(g * L, L)] = jnp.where(lm, key,
                                                jnp.int32(-0x80000000))

        def _count_gt(pivot):
            @plsc.parallel_loop(0, nv, carry=zero16)
            def _cnt(g, acc):
                kv = keybuf[pl.ds(g * L, L)]
                return acc + plsc.all_reduce_population_count(kv > pivot)
            return jnp.max(_cnt)

        def _bs_cond(s):
            lo, hi = s
            return hi - lo > 1

        def _bs_body(s):
            lo, hi = s
            mid = lo + ((hi - lo) >> 1)
            big = _count_gt(mid) >= K
            return (jnp.where(big, mid, lo), jnp.where(big, hi, mid))

        _, vstar = lax.while_loop(_bs_cond, _bs_body, (key_lo - 1, key_hi))
        g_cnt = _count_gt(vstar)
        b_allow = K - g_cnt

        jsplat = zero16 + j

        @plsc.parallel_loop(0, nv, carry=(zero16, zero16))
        def _sel(g, carry):
            ptrv, tiev = carry
            kv = keybuf[pl.ds(g * L, L)]
            gtm = kv > vstar
            eqm = kv == vstar
            cum_t = tiev + plsc.cumsum(eqm.astype(jnp.int32))
            selm = gtm | (eqm & (cum_t <= b_allow))
            posn = ptrv + plsc.cumsum(selm.astype(jnp.int32)) - 1
            vals = plsc.bitcast(jnp.where(kv < 0, kv ^ jnp.int32(0x7FFFFFFF),
                                          kv), jnp.float32)
            iv = idxbuf[pl.ds(g * L, L)]
            plsc.store_scatter(outidx, [jsplat, posn], iv, mask=selm)
            plsc.store_scatter(outval, [jsplat, posn], vals, mask=selm)
            return (ptrv + plsc.all_reduce_population_count(selm),
                    tiev + plsc.all_reduce_population_count(eqm))

    @pl.loop(0, RPW)
    def _pe_loop(j):
        pltpu.sync_copy(pos_hbm.at[outidx.at[j]], posbuf)
        ov = outval.at[j]

        @pl.loop(0, K // L)
        def _add(u):
            s = pl.ds(u * L, L)
            ov[s] = ov[s] + posbuf[s]

    pltpu.sync_copy(outval, out_hbm.at[pl.ds(wid * RPW, RPW)])


@jax.jit
def _run(x2, posf):
    f = pl.kernel(
        _sc_body,
        out_type=jax.ShapeDtypeStruct((R, K), jnp.float32),
        mesh=_MESH,
        scratch_types=[
            pltpu.VMEM((2 * N,), jnp.float32),
            pltpu.VMEM((CAP,), jnp.int32),
            pltpu.VMEM((CAP,), jnp.int32),
            pltpu.VMEM((RPW, K), jnp.float32),
            pltpu.VMEM((RPW, K), jnp.int32),
            pltpu.VMEM((K,), jnp.float32),
            pltpu.SemaphoreType.DMA,
        ],
        compiler_params=pltpu.CompilerParams(needs_layout_passes=False, use_tc_tiling_on_sc=True),
    )
    return f(x2, posf)


def kernel(x, pos_enc):
    b, c, h, w = x.shape
    out = _run(x.reshape(b * c, h * w), pos_enc.reshape(h * w))
    return out.reshape(b, c, K)
